# baseline (speedup 1.0000x reference)
import functools
import jax
import jax.numpy as jnp
from jax import lax
from jax.experimental import pallas as pl
from jax.experimental.pallas import tpu as pltpu
from jax.experimental.pallas import tpu_sc as plsc

F32 = jnp.float32
I32 = jnp.int32
HIGH = jax.lax.Precision.HIGHEST

N_WKR = 2000
N_TSK = 8000
R = 10
D = 256
E = 160000

NC = 2
NS = 16
NT = NC * NS
E_PAD = 163840
EPT = E_PAD // NT
IBLK = 16384
NBLK = E_PAD // IBLK
GB = 128
KSHIFT = 13
KMASK = (1 << KSHIFT) - 1

WKR_PAD = 2048
TSK_PAD = 8192
WROWS = WKR_PAD // NT
TROWS = TSK_PAD // NT


def _mm_a_body(lab_ref, w_ref, out_ref):
    out_ref[0] = jnp.dot(lab_ref[...], w_ref[0],
                         preferred_element_type=F32, precision=HIGH)


def _transform_tasks(labels, weight_worker):
    tb = 2000
    return pl.pallas_call(
        _mm_a_body,
        grid=(R, N_TSK // tb),
        in_specs=[
            pl.BlockSpec((tb, D), lambda r, t: (t, 0)),
            pl.BlockSpec((1, D, D), lambda r, t: (r, 0, 0)),
        ],
        out_specs=pl.BlockSpec((1, tb, D), lambda r, t: (r, t, 0)),
        out_shape=jax.ShapeDtypeStruct((R, N_TSK, D), F32),
    )(labels, weight_worker)


def _mm_c_body(accA_ref, degW_ref, w_ref, abil_ref, hw_ref):
    deg = jnp.maximum(jnp.sum(degW_ref[...], axis=0), 1.0)[:, None]
    ability = accA_ref[...] / deg
    abil_ref[...] = ability
    hw_ref[0] = jnp.dot(ability, w_ref[0],
                        preferred_element_type=F32, precision=HIGH)


def _transform_workers(accA, degW_p, weight_task):
    return pl.pallas_call(
        _mm_c_body,
        grid=(R,),
        in_specs=[
            pl.BlockSpec((WKR_PAD, D), lambda r: (0, 0)),
            pl.BlockSpec((NT, WKR_PAD), lambda r: (0, 0)),
            pl.BlockSpec((1, D, D), lambda r: (r, 0, 0)),
        ],
        out_specs=[
            pl.BlockSpec((WKR_PAD, D), lambda r: (0, 0)),
            pl.BlockSpec((1, WKR_PAD, D), lambda r: (r, 0, 0)),
        ],
        out_shape=[
            jax.ShapeDtypeStruct((WKR_PAD, D), F32),
            jax.ShapeDtypeStruct((R, WKR_PAD, D), F32),
        ],
    )(accA, degW_p, weight_task)


def _norm_body(acc_ref, degT_ref, out_ref):
    deg = jnp.maximum(jnp.sum(degT_ref[...], axis=0), 1.0)[:, None]
    out_ref[...] = acc_ref[...] / deg


def _normalize_tasks(acc2, degT_p):
    nb = 8
    rb = TSK_PAD // nb
    return pl.pallas_call(
        _norm_body,
        grid=(nb,),
        in_specs=[
            pl.BlockSpec((rb, D), lambda i: (i, 0)),
            pl.BlockSpec((NT, rb), lambda i: (0, i)),
        ],
        out_specs=pl.BlockSpec((rb, D), lambda i: (i, 0)),
        out_shape=jax.ShapeDtypeStruct((TSK_PAD, D), F32),
    )(acc2, degT_p)


def _sc_mesh():
    return plsc.VectorSubcoreMesh(core_axis_name="c", subcore_axis_name="s",
                                  num_cores=NC, num_subcores=NS)


def _zero_1d(ref):
    z = jnp.zeros((16,), F32)
    def body(i, _):
        ref[pl.ds(i * 16, 16)] = z
        return 0
    lax.fori_loop(0, ref.shape[0] // 16, body, 0)


def _degrees_sc(esrc, edst):
    @functools.partial(
        pl.kernel,
        out_type=[
            jax.ShapeDtypeStruct((NT, WKR_PAD), F32),
            jax.ShapeDtypeStruct((NT, TSK_PAD), F32),
        ],
        mesh=_sc_mesh(),
        compiler_params=pltpu.CompilerParams(needs_layout_passes=False),
        scratch_types=[
            pltpu.VMEM((EPT,), I32),
            pltpu.VMEM((EPT,), I32),
            pltpu.VMEM((WKR_PAD,), F32),
            pltpu.VMEM((TSK_PAD,), F32),
        ],
    )
    def k(esrc_h, edst_h, degW_o, degT_o, srcv, dstv, degW, degT):
        cid = lax.axis_index("c")
        sid = lax.axis_index("s")
        wid = sid * NC + cid
        _zero_1d(degW)
        _zero_1d(degT)
        pltpu.sync_copy(esrc_h.at[pl.ds(wid * EPT, EPT)], srcv)
        pltpu.sync_copy(edst_h.at[pl.ds(wid * EPT, EPT)], dstv)
        ones16 = jnp.ones((16,), F32)

        def body(i, _):
            s = srcv[pl.ds(i * 16, 16)]
            d = dstv[pl.ds(i * 16, 16)]
            plsc.addupdate_scatter(degW, [d], ones16)
            plsc.addupdate_scatter(degT, [s], ones16)
            return 0
        lax.fori_loop(0, EPT // 16, body, 0)
        pltpu.sync_copy(degW, degW_o.at[wid])
        pltpu.sync_copy(degT, degT_o.at[wid])

    return k(esrc, edst)


def _segsum_sc(packed, table, zbig, n_keys, krows):
    @functools.partial(
        pl.kernel,
        out_type=jax.ShapeDtypeStruct((n_keys, D), F32),
        mesh=_sc_mesh(),
        compiler_params=pltpu.CompilerParams(needs_layout_passes=False),
        scratch_types=[
            pltpu.VMEM((IBLK,), I32),
            pltpu.VMEM((160,), I32),
            pltpu.VMEM((160,), I32),
            pltpu.VMEM((GB, D), F32),
            pltpu.VMEM((krows, D), F32),
            pltpu.SemaphoreType.DMA,
        ],
    )
    def k(packed_h, table_h, zbig_h, out_h,
          pbuf, pend_g, pend_k, rows, acc, sem):
        cid = lax.axis_index("c")
        sid = lax.axis_index("s")
        wid = sid * NC + cid
        lo = wid * krows

        pltpu.sync_copy(zbig_h.at[pl.ds(0, krows)], acc)
        z16 = jnp.zeros((16,), I32)
        for j in range(10):
            pend_g[pl.ds(j * 16, 16)] = z16
            pend_k[pl.ds(j * 16, 16)] = z16

        def accumulate(n):
            pltpu.async_copy(table_h.at[pend_g.at[pl.ds(0, GB)]],
                             rows, sem).wait()

            def edge(e, _):
                kk = pend_k[pl.ds(e, 16)][0]
                for j in range(D // 16):
                    sl = pl.ds(j * 16, 16)
                    acc[kk, sl] = acc[kk, sl] + rows[e, sl]
                return 0
            lax.fori_loop(0, n, edge, 0)

        def block(bi, fill):
            pltpu.sync_copy(packed_h.at[pl.ds(bi * IBLK, IBLK)], pbuf)

            def sub(si, fill):
                w = pbuf[pl.ds(si * 16, 16)]
                key = jnp.bitwise_and(w, KMASK)
                g = jnp.right_shift(w, KSHIFT)
                m = jnp.logical_and(key >= lo, key < lo + krows)
                plsc.store_compressed(pend_g.at[pl.ds(fill, 16)], g, mask=m)
                plsc.store_compressed(pend_k.at[pl.ds(fill, 16)],
                                      key - lo, mask=m)
                fill = fill + jnp.sum(m.astype(I32))

                @pl.when(fill >= GB)
                def _():
                    accumulate(GB)
                    for j in range(2):
                        sl_to = pl.ds(j * 16, 16)
                        sl_from = pl.ds(GB + j * 16, 16)
                        pend_g[sl_to] = pend_g[sl_from]
                        pend_k[sl_to] = pend_k[sl_from]
                return jnp.where(fill >= GB, fill - GB, fill)
            return lax.fori_loop(0, IBLK // 16, sub, fill)

        fill = lax.fori_loop(0, NBLK, block, jnp.int32(0))

        @pl.when(fill > 0)
        def _():
            accumulate(fill)

        pltpu.sync_copy(acc, out_h.at[pl.ds(lo, krows)])

    return k(packed, table, zbig)


@jax.jit
def kernel(labels, edge_src_task, edge_dst_wkr, edge_type,
           weight_worker, weight_task):
    npad = E_PAD - E
    esrc = jnp.concatenate(
        [edge_src_task.astype(I32), jnp.full((npad,), N_TSK, I32)])
    edst = jnp.concatenate(
        [edge_dst_wkr.astype(I32), jnp.full((npad,), N_WKR, I32)])
    etyp = jnp.concatenate(
        [edge_type.astype(I32), jnp.zeros((npad,), I32)])
    zbig = jnp.zeros((TSK_PAD, D), F32)

    gidx1 = etyp * N_TSK + jnp.minimum(esrc, N_TSK - 1)
    packed1 = jnp.left_shift(gidx1, KSHIFT) | edst
    gidx2 = etyp * WKR_PAD + jnp.minimum(edst, WKR_PAD - 1)
    packed2 = jnp.left_shift(gidx2, KSHIFT) | esrc

    degW_p, degT_p = _degrees_sc(esrc, edst)
    ht = _transform_tasks(labels, weight_worker).reshape(R * N_TSK, D)
    accA = _segsum_sc(packed1, ht, zbig, WKR_PAD, WROWS)
    ability_full, hw = _transform_workers(accA, degW_p, weight_task)
    acc2 = _segsum_sc(packed2, hw.reshape(R * WKR_PAD, D), zbig,
                      TSK_PAD, TROWS)
    new_full = _normalize_tasks(acc2, degT_p)
    return ability_full[:N_WKR], new_full[:N_TSK]


# speedup vs baseline: 2.0822x; 2.0822x over previous
import functools
import jax
import jax.numpy as jnp
from jax import lax
from jax.experimental import pallas as pl
from jax.experimental.pallas import tpu as pltpu
from jax.experimental.pallas import tpu_sc as plsc

F32 = jnp.float32
I32 = jnp.int32
HIGH = jax.lax.Precision.HIGHEST

N_WKR = 2000
N_TSK = 8000
R = 10
D = 256
E = 160000

NC = 2
NS = 16
NT = NC * NS
E_PAD = E
EPT = E_PAD // NT
IBLK = 16000
NBLK = E_PAD // IBLK
GB = 128
KSHIFT = 13
KMASK = (1 << KSHIFT) - 1

WKR_PAD = 2048
TSK_PAD = 8192
WROWS = WKR_PAD // NT
TROWS = TSK_PAD // NT


def _mm_a_body(lab_ref, w_ref, out_ref):
    out_ref[0] = jnp.dot(lab_ref[...], w_ref[0],
                         preferred_element_type=F32)


def _transform_tasks(labels, weight_worker):
    tb = 2000
    return pl.pallas_call(
        _mm_a_body,
        grid=(R, N_TSK // tb),
        in_specs=[
            pl.BlockSpec((tb, D), lambda r, t: (t, 0)),
            pl.BlockSpec((1, D, D), lambda r, t: (r, 0, 0)),
        ],
        out_specs=pl.BlockSpec((1, tb, D), lambda r, t: (r, t, 0)),
        out_shape=jax.ShapeDtypeStruct((R, N_TSK, D), F32),
    )(labels, weight_worker)


def _mm_c_body(accA_ref, degW_ref, w_ref, abil_ref, hw_ref):
    deg = jnp.maximum(jnp.sum(degW_ref[...], axis=0), 1.0)[:, None]
    ability = accA_ref[...] / deg
    abil_ref[...] = ability
    hw_ref[0] = jnp.dot(ability, w_ref[0],
                        preferred_element_type=F32, precision=HIGH)


def _transform_workers(accA, degW_p, weight_task):
    return pl.pallas_call(
        _mm_c_body,
        grid=(R,),
        in_specs=[
            pl.BlockSpec((WKR_PAD, D), lambda r: (0, 0)),
            pl.BlockSpec((NT, WKR_PAD), lambda r: (0, 0)),
            pl.BlockSpec((1, D, D), lambda r: (r, 0, 0)),
        ],
        out_specs=[
            pl.BlockSpec((WKR_PAD, D), lambda r: (0, 0)),
            pl.BlockSpec((1, WKR_PAD, D), lambda r: (r, 0, 0)),
        ],
        out_shape=[
            jax.ShapeDtypeStruct((WKR_PAD, D), F32),
            jax.ShapeDtypeStruct((R, WKR_PAD, D), F32),
        ],
    )(accA, degW_p, weight_task)


def _norm_body(acc_ref, degT_ref, out_ref):
    deg = jnp.maximum(jnp.sum(degT_ref[...], axis=0), 1.0)[:, None]
    out_ref[...] = acc_ref[...] / deg


def _normalize_tasks(acc2, degT_p):
    nb = 8
    rb = TSK_PAD // nb
    return pl.pallas_call(
        _norm_body,
        grid=(nb,),
        in_specs=[
            pl.BlockSpec((rb, D), lambda i: (i, 0)),
            pl.BlockSpec((NT, rb), lambda i: (0, i)),
        ],
        out_specs=pl.BlockSpec((rb, D), lambda i: (i, 0)),
        out_shape=jax.ShapeDtypeStruct((TSK_PAD, D), F32),
    )(acc2, degT_p)


def _sc_mesh():
    return plsc.VectorSubcoreMesh(core_axis_name="c", subcore_axis_name="s",
                                  num_cores=NC, num_subcores=NS)


def _zero_1d(ref):
    z = jnp.zeros((16,), F32)
    def body(i, _):
        ref[pl.ds(i * 16, 16)] = z
        return 0
    lax.fori_loop(0, ref.shape[0] // 16, body, 0)


def _degrees_sc(esrc, edst):
    @functools.partial(
        pl.kernel,
        out_type=[
            jax.ShapeDtypeStruct((NT, WKR_PAD), F32),
            jax.ShapeDtypeStruct((NT, TSK_PAD), F32),
        ],
        mesh=_sc_mesh(),
        compiler_params=pltpu.CompilerParams(needs_layout_passes=False),
        scratch_types=[
            pltpu.VMEM((EPT + 16,), I32),
            pltpu.VMEM((EPT + 16,), I32),
            pltpu.VMEM((WKR_PAD,), F32),
            pltpu.VMEM((TSK_PAD,), F32),
        ],
    )
    def k(esrc_h, edst_h, degW_o, degT_o, srcv, dstv, degW, degT):
        cid = lax.axis_index("c")
        sid = lax.axis_index("s")
        wid = sid * NC + cid
        _zero_1d(degW)
        _zero_1d(degT)
        pltpu.sync_copy(esrc_h.at[pl.ds(wid * EPT, EPT)],
                        srcv.at[pl.ds(0, EPT)])
        pltpu.sync_copy(edst_h.at[pl.ds(wid * EPT, EPT)],
                        dstv.at[pl.ds(0, EPT)])
        ones16 = jnp.ones((16,), F32)

        def body(i, _):
            s = srcv[pl.ds(i * 16, 16)]
            d = dstv[pl.ds(i * 16, 16)]
            plsc.addupdate_scatter(degW, [d], ones16)
            plsc.addupdate_scatter(degT, [s], ones16)
            return 0
        lax.fori_loop(0, EPT // 16, body, 0)
        tail = EPT - (EPT // 16) * 16
        if tail:
            tm = jax.lax.iota(I32, 16) < tail
            ts = jnp.minimum(srcv[pl.ds(EPT - tail, 16)], TSK_PAD - 1)
            td = jnp.minimum(dstv[pl.ds(EPT - tail, 16)], WKR_PAD - 1)
            plsc.addupdate_scatter(degW, [td], ones16, mask=tm)
            plsc.addupdate_scatter(degT, [ts], ones16, mask=tm)
        pltpu.sync_copy(degW, degW_o.at[wid])
        pltpu.sync_copy(degT, degT_o.at[wid])

    return k(esrc, edst)


def _segsum_sc(packed, table, zbig, n_keys, krows):
    @functools.partial(
        pl.kernel,
        out_type=jax.ShapeDtypeStruct((n_keys, D), F32),
        mesh=_sc_mesh(),
        compiler_params=pltpu.CompilerParams(needs_layout_passes=False),
        scratch_types=[
            pltpu.VMEM((IBLK,), I32),
            pltpu.VMEM((288,), I32),
            pltpu.VMEM((288,), I32),
            pltpu.VMEM((GB,), I32),
            pltpu.VMEM((GB + 16,), I32),
            pltpu.VMEM((GB, D), F32),
            pltpu.VMEM((krows + 8, D), F32),
            pltpu.SemaphoreType.DMA,
        ],
    )
    def k(packed_h, table_h, zbig_h, out_h,
          pbuf, pend_g, pend_k, gstage, kstage, rows, acc, sem):
        cid = lax.axis_index("c")
        sid = lax.axis_index("s")
        wid = sid * NC + cid
        lo = wid * krows

        pltpu.sync_copy(zbig_h.at[pl.ds(0, krows)], acc.at[pl.ds(0, krows)])
        pltpu.sync_copy(zbig_h.at[pl.ds(0, 8)],
                        acc.at[pl.ds(krows, 8)])
        z16 = jnp.zeros((16,), I32)
        for j in range(288 // 16):
            pend_g[pl.ds(j * 16, 16)] = z16
            pend_k[pl.ds(j * 16, 16)] = z16

        col = [jax.lax.iota(I32, 16) + j * 16 for j in range(D // 16)]

        def accum_batch(ngroups):
            def grp(g, _):
                kv = kstage[pl.ds(g * 16, 16)]
                for e in range(16):
                    ks = jnp.full((16,), kv[e], I32)
                    for j in range(D // 16):
                        plsc.addupdate_scatter(
                            acc, [ks, col[j]],
                            rows[g * 16 + e, pl.ds(j * 16, 16)])
                return 0
            lax.fori_loop(0, ngroups, grp, 0)

        def snapshot():
            for j in range(GB // 16):
                sl = pl.ds(j * 16, 16)
                gstage[sl] = pend_g[sl]
                kstage[sl] = pend_k[sl]

        def wait_batch():
            pltpu.make_async_copy(table_h.at[pl.ds(0, GB)], rows,
                                  sem).wait()

        def block(bi, carry):
            pltpu.sync_copy(packed_h.at[pl.ds(bi * IBLK, IBLK)], pbuf)

            def sub(si, carry):
                fill, nfl = carry
                gs, ks, ms, cs = [], [], [], []
                for u in range(8):
                    w = pbuf[pl.ds(si * 128 + u * 16, 16)]
                    key = jnp.bitwise_and(w, KMASK)
                    m = jnp.logical_and(key >= lo, key < lo + krows)
                    gs.append(jnp.right_shift(w, KSHIFT))
                    ks.append(key - lo)
                    ms.append(m)
                    cs.append(plsc.all_reduce_population_count(m)[0])
                for u in range(8):
                    plsc.store_compressed(pend_g.at[pl.ds(fill, 16)],
                                          gs[u], mask=ms[u])
                    plsc.store_compressed(pend_k.at[pl.ds(fill, 16)],
                                          ks[u], mask=ms[u])
                    fill = fill + cs[u]

                flush = fill >= GB

                @pl.when(flush)
                def _():
                    @pl.when(nfl > 0)
                    def _():
                        wait_batch()
                        accum_batch(GB // 16)
                    snapshot()
                    pltpu.async_copy(table_h.at[gstage], rows, sem)
                    for j in range(10):
                        sl_to = pl.ds(j * 16, 16)
                        sl_from = pl.ds(GB + j * 16, 16)
                        pend_g[sl_to] = pend_g[sl_from]
                        pend_k[sl_to] = pend_k[sl_from]
                return (jnp.where(flush, fill - GB, fill),
                        jnp.where(flush, 1, nfl))
            return lax.fori_loop(0, IBLK // 128, sub, carry)

        fill, nfl = lax.fori_loop(0, NBLK, block,
                                  (jnp.int32(0), jnp.int32(0)))

        @pl.when(nfl > 0)
        def _():
            wait_batch()
            accum_batch(GB // 16)

        @pl.when(fill > 0)
        def _():
            snapshot()
            kstage[pl.ds(fill, 16)] = jnp.full((16,), krows, I32)
            pltpu.async_copy(table_h.at[gstage], rows, sem).wait()
            accum_batch((fill + 15) // 16)

        pltpu.sync_copy(acc.at[pl.ds(0, krows)], out_h.at[pl.ds(lo, krows)])

    return k(packed, table, zbig)


@jax.jit
def kernel(labels, edge_src_task, edge_dst_wkr, edge_type,
           weight_worker, weight_task):
    esrc = edge_src_task.astype(I32)
    edst = edge_dst_wkr.astype(I32)
    etyp = edge_type.astype(I32)
    zbig = jnp.zeros((TSK_PAD, D), F32)

    gidx1 = etyp * N_TSK + jnp.minimum(esrc, N_TSK - 1)
    packed1 = jnp.left_shift(gidx1, KSHIFT) | edst
    gidx2 = etyp * WKR_PAD + jnp.minimum(edst, WKR_PAD - 1)
    packed2 = jnp.left_shift(gidx2, KSHIFT) | esrc

    degW_p, degT_p = _degrees_sc(esrc, edst)
    ht = _transform_tasks(labels, weight_worker).reshape(R * N_TSK, D)
    accA = _segsum_sc(packed1, ht, zbig, WKR_PAD, WROWS)
    ability_full, hw = _transform_workers(accA, degW_p, weight_task)
    acc2 = _segsum_sc(packed2, hw.reshape(R * WKR_PAD, D), zbig,
                      TSK_PAD, TROWS)
    new_full = _normalize_tasks(acc2, degT_p)
    return ability_full[:N_WKR], new_full[:N_TSK]
